# baseline (speedup 1.0000x reference)
import jax
import jax.numpy as jnp
from jax import lax
from jax.experimental import pallas as pl
from jax.experimental.pallas import tpu as pltpu
from jax.experimental.pallas import tpu_sc as plsc

BATCH = 16384
EMB = 32
NC = 2
NS = 16
NW = NC * NS
B_PER_W = BATCH // NW
CHUNK = 128
NCHUNK = B_PER_W // CHUNK


def _sc_gather_body(d_idx, c0_idx, c1_idx, d_tab, c0_tab, c1_tab,
                    d_out, c0_out, c1_out,
                    idx_d, idx_c0, idx_c1, rows_d, rows_c0, rows_c1, sem):
    wid = lax.axis_index("s") * NC + lax.axis_index("c")
    base = wid * B_PER_W
    sl = pl.ds(base, B_PER_W)
    pltpu.sync_copy(d_idx.at[sl], idx_d)
    pltpu.sync_copy(c0_idx.at[sl], idx_c0)
    pltpu.sync_copy(c1_idx.at[sl], idx_c1)
    copies = []
    for tab, idx_v, rows_v in ((d_tab, idx_d, rows_d),
                               (c0_tab, idx_c0, rows_c0),
                               (c1_tab, idx_c1, rows_c1)):
        for k in range(NCHUNK):
            csl = pl.ds(k * CHUNK, CHUNK)
            copies.append(
                pltpu.async_copy(tab.at[idx_v.at[csl]], rows_v.at[csl], sem))
    for c in copies:
        c.wait()
    pltpu.sync_copy(rows_d, d_out.at[sl])
    pltpu.sync_copy(rows_c0, c0_out.at[sl])
    pltpu.sync_copy(rows_c1, c1_out.at[sl])


def _make_sc_gather():
    return pl.kernel(
        _sc_gather_body,
        mesh=plsc.VectorSubcoreMesh(core_axis_name="c", subcore_axis_name="s"),
        compiler_params=pltpu.CompilerParams(use_tc_tiling_on_sc=False),
        out_type=[jax.ShapeDtypeStruct((BATCH, EMB), jnp.float32)] * 3,
        scratch_types=[
            pltpu.VMEM((B_PER_W,), jnp.int32),
            pltpu.VMEM((B_PER_W,), jnp.int32),
            pltpu.VMEM((B_PER_W,), jnp.int32),
            pltpu.VMEM((B_PER_W, EMB), jnp.float32),
            pltpu.VMEM((B_PER_W, EMB), jnp.float32),
            pltpu.VMEM((B_PER_W, EMB), jnp.float32),
            pltpu.SemaphoreType.DMA,
        ],
    )

BB = 512


def _tc_body(didx, c0idx, c1idx, d_ref, c0_ref, c1_ref,
             w0, b0, w1, b1, w2, b2, o0, o1, o2):
    md = (didx[...] != 0).astype(jnp.float32)
    m0 = (c0idx[...] != 0).astype(jnp.float32)
    m1 = (c1idx[...] != 0).astype(jnp.float32)
    d = d_ref[...] * md
    ch = c0_ref[...] * m0 + c1_ref[...] * m1
    e = d * (ch * 0.5)
    dn = (((1,), (1,)), ((), ()))
    o0[...] = lax.dot_general(e, w0[...], dn,
                              preferred_element_type=jnp.float32) + b0[...]
    o1[...] = lax.dot_general(e, w1[...], dn,
                              preferred_element_type=jnp.float32) + b1[...]
    o2[...] = lax.dot_general(e, w2[...], dn,
                              preferred_element_type=jnp.float32) + b2[...]


def _tc_call(didx, c0idx, c1idx, d_rows, c0_rows, c1_rows,
             W0, b0, W1, b1, W2, b2):
    t0, t1, t2 = W0.shape[0], W1.shape[0], W2.shape[0]
    f32 = jnp.float32
    emb_spec = pl.BlockSpec((BB, EMB), lambda i: (i, 0))
    idx_spec = pl.BlockSpec((BB, 1), lambda i: (i, 0))
    full = lambda shape: pl.BlockSpec(shape, lambda i: (0, 0))
    return pl.pallas_call(
        _tc_body,
        grid=(BATCH // BB,),
        in_specs=[idx_spec, idx_spec, idx_spec, emb_spec, emb_spec, emb_spec,
                  full((t0, EMB)), full((1, t0)),
                  full((t1, EMB)), full((1, t1)),
                  full((t2, EMB)), full((1, t2))],
        out_specs=[pl.BlockSpec((BB, t0), lambda i: (i, 0)),
                   pl.BlockSpec((BB, t1), lambda i: (i, 0)),
                   pl.BlockSpec((BB, t2), lambda i: (i, 0))],
        out_shape=[jax.ShapeDtypeStruct((BATCH, t0), f32),
                   jax.ShapeDtypeStruct((BATCH, t1), f32),
                   jax.ShapeDtypeStruct((BATCH, t2), f32)],
    )(didx, c0idx, c1idx, d_rows, c0_rows, c1_rows,
      W0, b0.reshape(1, t0), W1, b1.reshape(1, t1), W2, b2.reshape(1, t2))


def kernel(dialects, chars, d_emb0, c_emb0, c_emb1, W0, b0, W1, b1, W2, b2):
    d_idx = dialects[:, 0].astype(jnp.int32)
    c0_idx = chars[:, 0].astype(jnp.int32)
    c1_idx = chars[:, 1].astype(jnp.int32)
    d_rows, c0_rows, c1_rows = _make_sc_gather()(
        d_idx, c0_idx, c1_idx, d_emb0, c_emb0, c_emb1)
    return tuple(_tc_call(
        d_idx.reshape(BATCH, 1), c0_idx.reshape(BATCH, 1),
        c1_idx.reshape(BATCH, 1), d_rows, c0_rows, c1_rows,
        W0, b0, W1, b1, W2, b2))


# speedup vs baseline: 1.1701x; 1.1701x over previous
import jax
import jax.numpy as jnp
from jax import lax
from jax.experimental import pallas as pl
from jax.experimental.pallas import tpu as pltpu
from jax.experimental.pallas import tpu_sc as plsc

BATCH = 16384
EMB = 32
NC = 2
NS = 16
NW = NC * NS
B_PER_W = BATCH // NW
CHUNK = 128
NCHUNK = B_PER_W // CHUNK


def _sc_gather_body(d_idx, c0_idx, c1_idx, d_tab, c0_tab, c1_tab,
                    d_out, c0_out, c1_out,
                    idx_d, idx_c0, idx_c1, rows_d, rows_c0, rows_c1, sem):
    wid = lax.axis_index("s") * NC + lax.axis_index("c")
    base = wid * B_PER_W
    sl = pl.ds(base, B_PER_W)
    pltpu.sync_copy(d_idx.at[sl], idx_d)
    pltpu.sync_copy(c0_idx.at[sl], idx_c0)
    pltpu.sync_copy(c1_idx.at[sl], idx_c1)
    copies = []
    for tab, idx_v, rows_v in ((d_tab, idx_d, rows_d),
                               (c0_tab, idx_c0, rows_c0),
                               (c1_tab, idx_c1, rows_c1)):
        for k in range(NCHUNK):
            csl = pl.ds(k * CHUNK, CHUNK)
            copies.append(
                pltpu.async_copy(tab.at[idx_v.at[csl]], rows_v.at[csl], sem))
    for c in copies:
        c.wait()
    pltpu.sync_copy(rows_d, d_out.at[sl])
    pltpu.sync_copy(rows_c0, c0_out.at[sl])
    pltpu.sync_copy(rows_c1, c1_out.at[sl])


def _make_sc_gather():
    return pl.kernel(
        _sc_gather_body,
        mesh=plsc.VectorSubcoreMesh(core_axis_name="c", subcore_axis_name="s"),
        compiler_params=pltpu.CompilerParams(use_tc_tiling_on_sc=False),
        out_type=[jax.ShapeDtypeStruct((BATCH, EMB), jnp.float32)] * 3,
        scratch_types=[
            pltpu.VMEM((B_PER_W,), jnp.int32),
            pltpu.VMEM((B_PER_W,), jnp.int32),
            pltpu.VMEM((B_PER_W,), jnp.int32),
            pltpu.VMEM((B_PER_W, EMB), jnp.float32),
            pltpu.VMEM((B_PER_W, EMB), jnp.float32),
            pltpu.VMEM((B_PER_W, EMB), jnp.float32),
            pltpu.SemaphoreType.DMA,
        ],
    )


BB = 2048


def _tc_body(didx, c0idx, c1idx, dT_ref, c0T_ref, c1T_ref,
             w0t, b0, w1t, b1, w2t, b2, o0T, o1, o2T):
    md = (didx[...] != 0).astype(jnp.float32)
    m0 = (c0idx[...] != 0).astype(jnp.float32)
    m1 = (c1idx[...] != 0).astype(jnp.float32)
    ch = c0T_ref[...] * m0 + c1T_ref[...] * m1
    eT = (dT_ref[...] * md) * (ch * 0.5)
    dn = (((0,), (0,)), ((), ()))
    o0T[...] = lax.dot_general(w0t[...], eT, dn,
                               preferred_element_type=jnp.float32) + b0[...]
    o1[...] = lax.dot_general(eT, w1t[...], dn,
                              preferred_element_type=jnp.float32) + b1[...]
    o2T[...] = lax.dot_general(w2t[...], eT, dn,
                               preferred_element_type=jnp.float32) + b2[...]


def _tc_call(didx, c0idx, c1idx, dT_g, c0T_g, c1T_g,
             W0, b0, W1, b1, W2, b2):
    t0, t1, t2 = W0.shape[0], W1.shape[0], W2.shape[0]
    f32 = jnp.float32
    embT_spec = pl.BlockSpec((EMB, BB), lambda i: (0, i))
    idx_spec = pl.BlockSpec((1, BB), lambda i: (0, i))
    full = lambda shape: pl.BlockSpec(shape, lambda i: (0, 0))
    return pl.pallas_call(
        _tc_body,
        grid=(BATCH // BB,),
        in_specs=[idx_spec, idx_spec, idx_spec,
                  embT_spec, embT_spec, embT_spec,
                  full((EMB, t0)), full((t0, 1)),
                  full((EMB, t1)), full((1, t1)),
                  full((EMB, t2)), full((t2, 1))],
        out_specs=[pl.BlockSpec((t0, BB), lambda i: (0, i)),
                   pl.BlockSpec((BB, t1), lambda i: (i, 0)),
                   pl.BlockSpec((t2, BB), lambda i: (0, i))],
        out_shape=[jax.ShapeDtypeStruct((t0, BATCH), f32),
                   jax.ShapeDtypeStruct((BATCH, t1), f32),
                   jax.ShapeDtypeStruct((t2, BATCH), f32)],
    )(didx, c0idx, c1idx, dT_g, c0T_g, c1T_g,
      W0.T, b0.reshape(t0, 1), W1.T, b1.reshape(1, t1),
      W2.T, b2.reshape(t2, 1))


def kernel(dialects, chars, d_emb0, c_emb0, c_emb1, W0, b0, W1, b1, W2, b2):
    d_idx = dialects[:, 0].astype(jnp.int32)
    c0_idx = chars[:, 0].astype(jnp.int32)
    c1_idx = chars[:, 1].astype(jnp.int32)
    d_rows, c0_rows, c1_rows = _make_sc_gather()(
        d_idx, c0_idx, c1_idx, d_emb0, c_emb0, c_emb1)
    o0T, o1, o2T = _tc_call(
        d_idx.reshape(1, BATCH), c0_idx.reshape(1, BATCH),
        c1_idx.reshape(1, BATCH), d_rows.T, c0_rows.T, c1_rows.T,
        W0, b0, W1, b1, W2, b2)
    return (o0T.T, o1, o2T.T)
